# baseline (speedup 1.0000x reference)
import jax
import jax.numpy as jnp
from jax import lax
from jax.experimental import pallas as pl
from jax.experimental.pallas import tpu as pltpu
from jax.experimental.pallas import tpu_sc as plsc

B, L, D = 1024, 200, 128
NC, NS = 2, 16
NW = NC * NS
BPW = B // NW
BB = 64

_sc_mesh = plsc.VectorSubcoreMesh(core_axis_name="c", subcore_axis_name="s")


def _gather_body(lbl_hbm, table_hbm, g_hbm, idx_v, rows_v, gsem):
    wid = lax.axis_index("s") * NC + lax.axis_index("c")
    b0 = wid * BPW
    pltpu.sync_copy(lbl_hbm.at[pl.ds(b0, BPW)], idx_v)
    pltpu.async_copy(table_hbm.at[idx_v], rows_v, gsem).wait()
    pltpu.sync_copy(rows_v, g_hbm.at[pl.ds(b0, BPW)])


def _xcopy_body(x_ref, out_ref):
    out_ref[...] = x_ref[...]


def _emb_body(_, g_ref, out_ref):
    g = g_ref[...]
    out_ref[...] = jnp.broadcast_to(g[:, None, :], (BB, L, D))


@jax.jit
def kernel(x, labels_pointer, emb_table):
    gather = pl.kernel(
        _gather_body,
        out_type=jax.ShapeDtypeStruct((B, D), emb_table.dtype),
        mesh=_sc_mesh,
        scratch_types=[
            pltpu.VMEM((BPW,), jnp.int32),
            pltpu.VMEM((BPW, D), jnp.float32),
            pltpu.SemaphoreType.DMA,
        ],
    )
    tmp = pl.pallas_call(
        _xcopy_body,
        grid=(B // BB,),
        in_specs=[pl.BlockSpec((BB, L, D), lambda i: (i, 0, 0))],
        out_specs=pl.BlockSpec((BB, L, D), lambda i: (i, 0, 0)),
        out_shape=jax.ShapeDtypeStruct((B, L, 2 * D), x.dtype),
        compiler_params=pltpu.CompilerParams(
            dimension_semantics=("parallel",)),
    )(x)


    g = gather(labels_pointer, emb_table)

    return pl.pallas_call(
        _emb_body,
        grid=(B // BB,),
        in_specs=[
            pl.BlockSpec(memory_space=pltpu.MemorySpace.HBM),
            pl.BlockSpec((BB, D), lambda i: (i, 0)),
        ],
        out_specs=pl.BlockSpec((BB, L, D), lambda i: (i, 0, 1)),
        out_shape=jax.ShapeDtypeStruct((B, L, 2 * D), x.dtype),
        input_output_aliases={0: 0},
        compiler_params=pltpu.CompilerParams(
            dimension_semantics=("parallel",)),
    )(tmp, g)


# speedup vs baseline: 1.0171x; 1.0171x over previous
import jax
import jax.numpy as jnp
from jax import lax
from jax.experimental import pallas as pl
from jax.experimental.pallas import tpu as pltpu
from jax.experimental.pallas import tpu_sc as plsc

B, L, D = 1024, 200, 128
NC, NS = 2, 16
NW = NC * NS
BPW = B // NW
BB = 64

_sc_mesh = plsc.VectorSubcoreMesh(core_axis_name="c", subcore_axis_name="s")


def _gather_body(lbl_hbm, table_hbm, g_hbm, idx_v, rows_v, gsem):
    wid = lax.axis_index("s") * NC + lax.axis_index("c")
    b0 = wid * BPW
    pltpu.sync_copy(lbl_hbm.at[pl.ds(b0, BPW)], idx_v)
    pltpu.async_copy(table_hbm.at[idx_v], rows_v, gsem).wait()
    pltpu.sync_copy(rows_v, g_hbm.at[pl.ds(b0, BPW)])


def _xcopy_body(x_ref, out_ref):
    out_ref[...] = x_ref[...]


def _emb_body(_, g_ref, out_ref):
    g = g_ref[...]
    out_ref[...] = jnp.broadcast_to(g[:, None, :], (BB, L, D))


@jax.jit
def kernel(x, labels_pointer, emb_table):
    gather = pl.kernel(
        _gather_body,
        out_type=jax.ShapeDtypeStruct((B, D), emb_table.dtype),
        mesh=_sc_mesh,
        scratch_types=[
            pltpu.VMEM((BPW,), jnp.int32),
            pltpu.VMEM((BPW, D), jnp.float32),
            pltpu.SemaphoreType.DMA,
        ],
    )
    g = gather(labels_pointer, emb_table)

    tmp = pl.pallas_call(
        _xcopy_body,
        grid=(B // BB,),
        in_specs=[pl.BlockSpec((BB, L, D), lambda i: (i, 0, 0))],
        out_specs=pl.BlockSpec((BB, L, D), lambda i: (i, 0, 0)),
        out_shape=jax.ShapeDtypeStruct((B, L, 2 * D), x.dtype),
        compiler_params=pltpu.CompilerParams(
            dimension_semantics=("parallel",)),
    )(x)

    return pl.pallas_call(
        _emb_body,
        grid=(B // BB,),
        in_specs=[
            pl.BlockSpec(memory_space=pltpu.MemorySpace.HBM),
            pl.BlockSpec((BB, D), lambda i: (i, 0)),
        ],
        out_specs=pl.BlockSpec((BB, L, D), lambda i: (i, 0, 1)),
        out_shape=jax.ShapeDtypeStruct((B, L, 2 * D), x.dtype),
        input_output_aliases={0: 0},
        compiler_params=pltpu.CompilerParams(
            dimension_semantics=("parallel",)),
    )(tmp, g)
